# baseline (speedup 1.0000x reference)
import functools

import jax
import jax.numpy as jnp
from jax import lax
from jax.experimental import pallas as pl
from jax.experimental.pallas import tpu as pltpu
from jax.experimental.pallas import tpu_sc as plsc

N_NODES = 10000
N_EDGES = 320000
D = 128

NUM_WORKERS = 32
PER_WORKER = N_EDGES // NUM_WORKERS
CHUNK = 80
N_CHUNKS = PER_WORKER // CHUNK
GROUPS = CHUNK // 16

_MESH = plsc.VectorSubcoreMesh(core_axis_name="c", subcore_axis_name="s")

_GATHER_DNUMS = lax.GatherDimensionNumbers(
    offset_dims=(), collapsed_slice_dims=(0,), start_index_map=(0,))


def _perm(v, idx):
    return lax.gather(v, idx[:, None], _GATHER_DNUMS, slice_sizes=(1,),
                      mode=lax.GatherScatterMode.PROMISE_IN_BOUNDS)


@functools.partial(
    pl.kernel,
    out_type=(
        jax.ShapeDtypeStruct((N_EDGES,), jnp.float32),
        jax.ShapeDtypeStruct((N_EDGES,), jnp.float32),
    ),
    mesh=_MESH,
    scratch_types=[
        pltpu.VMEM((PER_WORKER,), jnp.int32),
        pltpu.VMEM((PER_WORKER,), jnp.int32),
        pltpu.VMEM((PER_WORKER,), jnp.int32),
        [pltpu.VMEM((CHUNK, D), jnp.float32) for _ in range(2)],
        [pltpu.VMEM((CHUNK, D), jnp.float32) for _ in range(2)],
        [pltpu.VMEM((CHUNK, D), jnp.float32) for _ in range(2)],
        pltpu.VMEM((CHUNK,), jnp.float32),
        pltpu.VMEM((CHUNK,), jnp.float32),
        [pltpu.SemaphoreType.DMA for _ in range(2)],
    ],
)
def _sc_scores(src_hbm, tar_hbm, negi_hbm, table_hbm, pos_hbm, neg_hbm,
               sidx, tidx, nidx, srows, trows, nrows, pbuf, nbuf, sems):
    wid = lax.axis_index("s") * 2 + lax.axis_index("c")
    base_w = wid * PER_WORKER
    lane = lax.iota(jnp.int32, 16)

    pltpu.sync_copy(src_hbm.at[pl.ds(base_w, PER_WORKER)], sidx)
    pltpu.sync_copy(tar_hbm.at[pl.ds(base_w, PER_WORKER)], tidx)
    pltpu.sync_copy(negi_hbm.at[pl.ds(base_w, PER_WORKER)], nidx)

    def issue(c, slot):
        off = c * CHUNK
        pltpu.async_copy(table_hbm.at[sidx.at[pl.ds(off, CHUNK)]],
                         srows[slot], sems[slot])
        pltpu.async_copy(table_hbm.at[tidx.at[pl.ds(off, CHUNK)]],
                         trows[slot], sems[slot])
        pltpu.async_copy(table_hbm.at[nidx.at[pl.ds(off, CHUNK)]],
                         nrows[slot], sems[slot])

    def drain(c, slot):
        off = c * CHUNK
        pltpu.make_async_copy(table_hbm.at[sidx.at[pl.ds(off, CHUNK)]],
                              srows[slot], sems[slot]).wait()
        pltpu.make_async_copy(table_hbm.at[tidx.at[pl.ds(off, CHUNK)]],
                              trows[slot], sems[slot]).wait()
        pltpu.make_async_copy(table_hbm.at[nidx.at[pl.ds(off, CHUNK)]],
                              nrows[slot], sems[slot]).wait()

    def merge(a, b, sh):
        am = a + _perm(a, lane ^ sh)
        bm = b + _perm(b, lane ^ sh)
        return jnp.where((lane & sh) == 0, am, bm)

    def compute(c, slot):
        sr, tr, nr = srows[slot], trows[slot], nrows[slot]

        def group_body(g, carry):
            pstack = [None] * 4
            nstack = [None] * 4
            for k in range(16):
                e = g * 16 + k
                pp = []
                nn = []
                for j in range(D // 16):
                    s = sr[e, pl.ds(16 * j, 16)]
                    t = tr[e, pl.ds(16 * j, 16)]
                    n = nr[e, pl.ds(16 * j, 16)]
                    pp.append(s * t)
                    nn.append(s * n)
                while len(pp) > 1:
                    pp = [pp[i] + pp[i + 1] for i in range(0, len(pp), 2)]
                    nn = [nn[i] + nn[i + 1] for i in range(0, len(nn), 2)]
                pa, na = pp[0], nn[0]
                if k % 2 == 0:
                    pstack[0], nstack[0] = pa, na
                else:
                    pm = merge(pstack[0], pa, 1)
                    nm = merge(nstack[0], na, 1)
                    lvl = 1
                    while lvl < 4 and (k >> lvl) & 1:
                        pm = merge(pstack[lvl], pm, 1 << lvl)
                        nm = merge(nstack[lvl], nm, 1 << lvl)
                        lvl += 1
                    if lvl < 4:
                        pstack[lvl], nstack[lvl] = pm, nm
            pbuf[pl.ds(g * 16, 16)] = pm
            nbuf[pl.ds(g * 16, 16)] = nm
            return carry

        lax.fori_loop(0, GROUPS, group_body, 0)
        base = base_w + c * CHUNK
        pltpu.sync_copy(pbuf, pos_hbm.at[pl.ds(base, CHUNK)])
        pltpu.sync_copy(nbuf, neg_hbm.at[pl.ds(base, CHUNK)])

    issue(0, 0)

    def chunk_pair(c2, carry):
        for b in range(2):
            c = 2 * c2 + b
            issue(c + 1, 1 - b)
            drain(c, b)
            compute(c, b)
        return carry

    lax.fori_loop(0, (N_CHUNKS - 1) // 2, chunk_pair, 0)
    drain(N_CHUNKS - 1, 0)
    compute(N_CHUNKS - 1, 0)


def _tc_finalize(pos_ref, neg_ref, table_ref, out_ref):
    pos = pos_ref[...]
    neg = neg_ref[...]
    pos_loss = jnp.mean(jax.nn.softplus(-pos))
    neg_loss = jnp.mean(jax.nn.softplus(neg))
    col_mean = jnp.mean(table_ref[...], axis=0)
    avg_loss = jnp.mean(jnp.log(col_mean + 0.0001))
    out_ref[0, 0] = pos_loss + neg_loss - avg_loss


def kernel(edges, cluster_logits):
    neg_idx = jax.random.randint(
        jax.random.key(42), (edges.shape[1],), 0, cluster_logits.shape[0],
        dtype=jnp.int32)
    src_ids = edges[0]
    tar_ids = edges[1]
    pos_score, neg_score = _sc_scores(src_ids, tar_ids, neg_idx,
                                      cluster_logits)
    out = pl.pallas_call(
        _tc_finalize,
        out_shape=jax.ShapeDtypeStruct((1, 1), jnp.float32),
        out_specs=pl.BlockSpec(memory_space=pltpu.SMEM),
    )(pos_score.reshape(2500, D), neg_score.reshape(2500, D),
      cluster_logits)
    return out[0, 0]


# speedup vs baseline: 1.0686x; 1.0686x over previous
import functools

import jax
import jax.numpy as jnp
from jax import lax
from jax.experimental import pallas as pl
from jax.experimental.pallas import tpu as pltpu
from jax.experimental.pallas import tpu_sc as plsc

N_NODES = 10000
N_EDGES = 320000
D = 128

NUM_WORKERS = 32
PER_WORKER = N_EDGES // NUM_WORKERS
CHUNK = 80
N_CHUNKS = PER_WORKER // CHUNK
GROUPS = CHUNK // 16

_MESH = plsc.VectorSubcoreMesh(core_axis_name="c", subcore_axis_name="s")

_GATHER_DNUMS = lax.GatherDimensionNumbers(
    offset_dims=(), collapsed_slice_dims=(0,), start_index_map=(0,))


def _perm(v, idx):
    return lax.gather(v, idx[:, None], _GATHER_DNUMS, slice_sizes=(1,),
                      mode=lax.GatherScatterMode.PROMISE_IN_BOUNDS)


@functools.partial(
    pl.kernel,
    out_type=(
        jax.ShapeDtypeStruct((N_EDGES,), jnp.float32),
        jax.ShapeDtypeStruct((N_EDGES,), jnp.float32),
    ),
    mesh=_MESH,
    scratch_types=[
        pltpu.VMEM((PER_WORKER,), jnp.int32),
        pltpu.VMEM((PER_WORKER,), jnp.int32),
        pltpu.VMEM((PER_WORKER,), jnp.int32),
        [pltpu.VMEM((CHUNK, D), jnp.float32) for _ in range(2)],
        [pltpu.VMEM((CHUNK, D), jnp.float32) for _ in range(2)],
        [pltpu.VMEM((CHUNK, D), jnp.float32) for _ in range(2)],
        pltpu.VMEM((CHUNK,), jnp.float32),
        pltpu.VMEM((CHUNK,), jnp.float32),
        [pltpu.SemaphoreType.DMA for _ in range(2)],
    ],
)
def _sc_scores(src_hbm, tar_hbm, negi_hbm, table_hbm, pos_hbm, neg_hbm,
               sidx, tidx, nidx, srows, trows, nrows, pbuf, nbuf, sems):
    wid = lax.axis_index("s") * 2 + lax.axis_index("c")
    base_w = wid * PER_WORKER
    lane = lax.iota(jnp.int32, 16)

    pltpu.sync_copy(src_hbm.at[pl.ds(base_w, PER_WORKER)], sidx)
    pltpu.sync_copy(tar_hbm.at[pl.ds(base_w, PER_WORKER)], tidx)
    pltpu.sync_copy(negi_hbm.at[pl.ds(base_w, PER_WORKER)], nidx)

    def issue(c, slot):
        off = c * CHUNK
        pltpu.async_copy(table_hbm.at[sidx.at[pl.ds(off, CHUNK)]],
                         srows[slot], sems[slot])
        pltpu.async_copy(table_hbm.at[tidx.at[pl.ds(off, CHUNK)]],
                         trows[slot], sems[slot])
        pltpu.async_copy(table_hbm.at[nidx.at[pl.ds(off, CHUNK)]],
                         nrows[slot], sems[slot])

    def drain(c, slot):
        off = c * CHUNK
        pltpu.make_async_copy(table_hbm.at[sidx.at[pl.ds(off, CHUNK)]],
                              srows[slot], sems[slot]).wait()
        pltpu.make_async_copy(table_hbm.at[tidx.at[pl.ds(off, CHUNK)]],
                              trows[slot], sems[slot]).wait()
        pltpu.make_async_copy(table_hbm.at[nidx.at[pl.ds(off, CHUNK)]],
                              nrows[slot], sems[slot]).wait()

    def merge(a, b, sh):
        am = a + _perm(a, lane ^ sh)
        bm = b + _perm(b, lane ^ sh)
        return jnp.where((lane & sh) == 0, am, bm)

    def compute(c, slot):
        sr, tr, nr = srows[slot], trows[slot], nrows[slot]

        def dot2(e):
            ap = [None, None]
            an = [None, None]
            for j in range(D // 16):
                s = sr[e, pl.ds(16 * j, 16)]
                t = tr[e, pl.ds(16 * j, 16)]
                n = nr[e, pl.ds(16 * j, 16)]
                pj = s * t
                nj = s * n
                b = j & 1
                ap[b] = pj if ap[b] is None else ap[b] + pj
                an[b] = nj if an[b] is None else an[b] + nj
            return ap[0] + ap[1], an[0] + an[1]

        def group_body(g, carry):
            pstack = [None] * 3
            nstack = [None] * 3
            for m in range(8):
                pa0, na0 = dot2(g * 16 + 2 * m)
                pa1, na1 = dot2(g * 16 + 2 * m + 1)
                pm = merge(pa0, pa1, 1)
                nm = merge(na0, na1, 1)
                lvl = 0
                while lvl < 3 and (m >> lvl) & 1:
                    pm = merge(pstack[lvl], pm, 2 << lvl)
                    nm = merge(nstack[lvl], nm, 2 << lvl)
                    lvl += 1
                if lvl < 3:
                    pstack[lvl], nstack[lvl] = pm, nm
            pbuf[pl.ds(g * 16, 16)] = pm
            nbuf[pl.ds(g * 16, 16)] = nm
            return carry

        lax.fori_loop(0, GROUPS, group_body, 0)
        base = base_w + c * CHUNK
        pltpu.sync_copy(pbuf, pos_hbm.at[pl.ds(base, CHUNK)])
        pltpu.sync_copy(nbuf, neg_hbm.at[pl.ds(base, CHUNK)])

    issue(0, 0)

    def chunk_pair(c2, carry):
        for b in range(2):
            c = 2 * c2 + b
            issue(c + 1, 1 - b)
            drain(c, b)
            compute(c, b)
        return carry

    lax.fori_loop(0, (N_CHUNKS - 1) // 2, chunk_pair, 0)
    drain(N_CHUNKS - 1, 0)
    compute(N_CHUNKS - 1, 0)


def _tc_finalize(pos_ref, neg_ref, table_ref, out_ref):
    pos = pos_ref[...]
    neg = neg_ref[...]
    pos_loss = jnp.mean(jax.nn.softplus(-pos))
    neg_loss = jnp.mean(jax.nn.softplus(neg))
    col_mean = jnp.mean(table_ref[...], axis=0)
    avg_loss = jnp.mean(jnp.log(col_mean + 0.0001))
    out_ref[0, 0] = pos_loss + neg_loss - avg_loss


def kernel(edges, cluster_logits):
    neg_idx = jax.random.randint(
        jax.random.key(42), (edges.shape[1],), 0, cluster_logits.shape[0],
        dtype=jnp.int32)
    src_ids = edges[0]
    tar_ids = edges[1]
    pos_score, neg_score = _sc_scores(src_ids, tar_ids, neg_idx,
                                      cluster_logits)
    out = pl.pallas_call(
        _tc_finalize,
        out_shape=jax.ShapeDtypeStruct((1, 1), jnp.float32),
        out_specs=pl.BlockSpec(memory_space=pltpu.SMEM),
    )(pos_score.reshape(2500, D), neg_score.reshape(2500, D),
      cluster_logits)
    return out[0, 0]


# speedup vs baseline: 1.2606x; 1.1797x over previous
import functools

import jax
import jax.numpy as jnp
from jax import lax
from jax.experimental import pallas as pl
from jax.experimental.pallas import tpu as pltpu
from jax.experimental.pallas import tpu_sc as plsc

N_NODES = 10000
N_EDGES = 320000
D = 128

NUM_WORKERS = 32
PER_WORKER = N_EDGES // NUM_WORKERS
CHUNK = 80
N_CHUNKS = PER_WORKER // CHUNK
GROUPS = CHUNK // 16

_MESH = plsc.VectorSubcoreMesh(core_axis_name="c", subcore_axis_name="s")

_GATHER_DNUMS = lax.GatherDimensionNumbers(
    offset_dims=(), collapsed_slice_dims=(0,), start_index_map=(0,))


def _perm(v, idx):
    return lax.gather(v, idx[:, None], _GATHER_DNUMS, slice_sizes=(1,),
                      mode=lax.GatherScatterMode.PROMISE_IN_BOUNDS)


@functools.partial(
    pl.kernel,
    out_type=(
        jax.ShapeDtypeStruct((N_EDGES,), jnp.float32),
        jax.ShapeDtypeStruct((N_EDGES,), jnp.float32),
    ),
    mesh=_MESH,
    scratch_types=[
        pltpu.VMEM((PER_WORKER,), jnp.int32),
        pltpu.VMEM((PER_WORKER,), jnp.int32),
        pltpu.VMEM((PER_WORKER,), jnp.int32),
        [pltpu.VMEM((CHUNK, D), jnp.float32) for _ in range(2)],
        [pltpu.VMEM((CHUNK, D), jnp.float32) for _ in range(2)],
        [pltpu.VMEM((CHUNK, D), jnp.float32) for _ in range(2)],
        pltpu.VMEM((CHUNK,), jnp.float32),
        pltpu.VMEM((CHUNK,), jnp.float32),
        [pltpu.SemaphoreType.DMA for _ in range(2)],
    ],
)
def _sc_scores(src_hbm, tar_hbm, negi_hbm, table_hbm, pos_hbm, neg_hbm,
               sidx, tidx, nidx, srows, trows, nrows, pbuf, nbuf, sems):
    wid = lax.axis_index("s") * 2 + lax.axis_index("c")
    base_w = wid * PER_WORKER
    lane = lax.iota(jnp.int32, 16)

    pltpu.sync_copy(src_hbm.at[pl.ds(base_w, PER_WORKER)], sidx)
    pltpu.sync_copy(tar_hbm.at[pl.ds(base_w, PER_WORKER)], tidx)
    pltpu.sync_copy(negi_hbm.at[pl.ds(base_w, PER_WORKER)], nidx)

    def issue(c, slot):
        off = c * CHUNK
        pltpu.async_copy(table_hbm.at[sidx.at[pl.ds(off, CHUNK)]],
                         srows[slot], sems[slot])
        pltpu.async_copy(table_hbm.at[tidx.at[pl.ds(off, CHUNK)]],
                         trows[slot], sems[slot])
        pltpu.async_copy(table_hbm.at[nidx.at[pl.ds(off, CHUNK)]],
                         nrows[slot], sems[slot])

    def drain(c, slot):
        off = c * CHUNK
        pltpu.make_async_copy(table_hbm.at[sidx.at[pl.ds(off, CHUNK)]],
                              srows[slot], sems[slot]).wait()
        pltpu.make_async_copy(table_hbm.at[tidx.at[pl.ds(off, CHUNK)]],
                              trows[slot], sems[slot]).wait()
        pltpu.make_async_copy(table_hbm.at[nidx.at[pl.ds(off, CHUNK)]],
                              nrows[slot], sems[slot]).wait()

    def merge(a, b, sh):
        am = a + _perm(a, lane ^ sh)
        bm = b + _perm(b, lane ^ sh)
        return jnp.where((lane & sh) == 0, am, bm)

    def compute(c, slot):
        sr, tr, nr = srows[slot], trows[slot], nrows[slot]

        zero16 = jnp.zeros((16,), jnp.float32)
        for g in range(GROUPS):
            pbuf[pl.ds(g * 16, 16)] = zero16
            nbuf[pl.ds(g * 16, 16)] = zero16

        @plsc.parallel_loop(0, CHUNK, unroll=4)
        def _edge(e):
            ap = [None, None]
            an = [None, None]
            for j in range(D // 16):
                s = sr[e, pl.ds(16 * j, 16)]
                t = tr[e, pl.ds(16 * j, 16)]
                n = nr[e, pl.ds(16 * j, 16)]
                pj = s * t
                nj = s * n
                b = j & 1
                ap[b] = pj if ap[b] is None else ap[b] + pj
                an[b] = nj if an[b] is None else an[b] + nj
            pa = ap[0] + ap[1]
            na = an[0] + an[1]
            for sh in (8, 4, 2, 1):
                perm = lane ^ sh
                pa = pa + _perm(pa, perm)
                na = na + _perm(na, perm)
            k = e & 15
            off = pl.multiple_of(e - k, 16)
            sel = lane == k
            plsc.addupdate(pbuf.at[pl.ds(off, 16)],
                           jnp.where(sel, pa, 0.0))
            plsc.addupdate(nbuf.at[pl.ds(off, 16)],
                           jnp.where(sel, na, 0.0))
        base = base_w + c * CHUNK
        pltpu.sync_copy(pbuf, pos_hbm.at[pl.ds(base, CHUNK)])
        pltpu.sync_copy(nbuf, neg_hbm.at[pl.ds(base, CHUNK)])

    issue(0, 0)

    def chunk_pair(c2, carry):
        for b in range(2):
            c = 2 * c2 + b
            issue(c + 1, 1 - b)
            drain(c, b)
            compute(c, b)
        return carry

    lax.fori_loop(0, (N_CHUNKS - 1) // 2, chunk_pair, 0)
    drain(N_CHUNKS - 1, 0)
    compute(N_CHUNKS - 1, 0)


def _tc_finalize(pos_ref, neg_ref, table_ref, out_ref):
    pos = pos_ref[...]
    neg = neg_ref[...]
    pos_loss = jnp.mean(jax.nn.softplus(-pos))
    neg_loss = jnp.mean(jax.nn.softplus(neg))
    col_mean = jnp.mean(table_ref[...], axis=0)
    avg_loss = jnp.mean(jnp.log(col_mean + 0.0001))
    out_ref[0, 0] = pos_loss + neg_loss - avg_loss


def kernel(edges, cluster_logits):
    neg_idx = jax.random.randint(
        jax.random.key(42), (edges.shape[1],), 0, cluster_logits.shape[0],
        dtype=jnp.int32)
    src_ids = edges[0]
    tar_ids = edges[1]
    pos_score, neg_score = _sc_scores(src_ids, tar_ids, neg_idx,
                                      cluster_logits)
    out = pl.pallas_call(
        _tc_finalize,
        out_shape=jax.ShapeDtypeStruct((1, 1), jnp.float32),
        out_specs=pl.BlockSpec(memory_space=pltpu.SMEM),
    )(pos_score.reshape(2500, D), neg_score.reshape(2500, D),
      cluster_logits)
    return out[0, 0]
